# baseline (speedup 1.0000x reference)
import functools

import jax
import jax.numpy as jnp
from jax import lax
from jax.experimental import pallas as pl
from jax.experimental.pallas import tpu as pltpu
from jax.experimental.pallas import tpu_sc as plsc

N_NODES = 100000
N_EDGES = 6400000

NC = 2
NS = 16
LANES = 16
NW = NC * NS

N_PAD = 102400
EPT = N_EDGES // NW
CHUNK = 1600
GSUB = 80
N_OWN = N_PAD // NW


def _lane_gather(vec, idx):
    return jnp.take(vec, idx, mode="wrap")


def _scatter_max_groups(agg_v, dst_v, val_v, n_groups):
    @pl.loop(0, n_groups)
    def _group(g):
        off = g * LANES
        d = dst_v[pl.ds(off, LANES)]
        v = val_v[pl.ds(off, LANES)]
        cur = plsc.load_gather(agg_v, [d])
        plsc.store_scatter(agg_v, [d], jnp.maximum(cur, v))
        chk = plsc.load_gather(agg_v, [d])
        pending = chk < v

        def _retry(p):
            cur = plsc.load_gather(agg_v, [d])
            plsc.store_scatter(agg_v, [d], jnp.maximum(cur, v), mask=p)
            chk = plsc.load_gather(agg_v, [d])
            return p & (chk < v)

        lax.while_loop(lambda p: jnp.any(p), _retry, pending)


def _phase1_body(xp_hbm, src_hbm, dst_hbm, part_hbm, agg_v, src_v, dst_v, val_v, sem):
    wid = lax.axis_index("s") * NC + lax.axis_index("c")
    pltpu.sync_copy(xp_hbm, agg_v)
    ebase = wid * EPT

    @pl.loop(0, EPT // CHUNK)
    def _chunk(c):
        base = ebase + c * CHUNK
        pltpu.sync_copy(src_hbm.at[pl.ds(base, CHUNK)], src_v)
        pltpu.sync_copy(dst_hbm.at[pl.ds(base, CHUNK)], dst_v)
        descs = []
        for j in range(CHUNK // GSUB):
            descs.append(pltpu.async_copy(
                xp_hbm.at[src_v.at[pl.ds(j * GSUB, GSUB)]],
                val_v.at[pl.ds(j * GSUB, GSUB)], sem))
        for desc in descs:
            desc.wait()
        _scatter_max_groups(agg_v, dst_v, val_v, CHUNK // LANES)

    pltpu.sync_copy(agg_v, part_hbm.at[wid])


def _phase2_body(xp_hbm, part_hbm, w0_hbm, w1_hbm, out_hbm,
                 acc_v, ld_v, xv_v, out_v, w0_vm, w1_vm):
    wid = lax.axis_index("s") * NC + lax.axis_index("c")
    base = wid * N_OWN
    pltpu.sync_copy(w0_hbm, w0_vm)
    pltpu.sync_copy(w1_hbm, w1_vm)
    pltpu.sync_copy(xp_hbm.at[pl.ds(base, N_OWN)], xv_v)
    pltpu.sync_copy(part_hbm.at[0, pl.ds(base, N_OWN)], acc_v)

    @pl.loop(1, NW)
    def _merge(t):
        pltpu.sync_copy(part_hbm.at[t, pl.ds(base, N_OWN)], ld_v)

        @pl.loop(0, N_OWN // LANES)
        def _vmax(g):
            off = g * LANES
            acc_v[pl.ds(off, LANES)] = jnp.maximum(
                acc_v[pl.ds(off, LANES)], ld_v[pl.ds(off, LANES)])

    w0 = w0_vm[...]
    w1 = w1_vm[...]

    @pl.loop(0, N_OWN // LANES)
    def _update(g):
        off = g * LANES
        z = w0 * xv_v[pl.ds(off, LANES)] + w1 * acc_v[pl.ds(off, LANES)]
        t = 1.0 - 2.0 / (jnp.exp(2.0 * z) + 1.0)
        e = jnp.exp(t - t)
        out_v[pl.ds(off, LANES)] = e / e

    pltpu.sync_copy(out_v, out_hbm.at[pl.ds(base, N_OWN)])


def kernel(x, edge_index, W):
    xf = x.reshape(N_NODES)
    xp = jnp.concatenate([xf, jnp.zeros((N_PAD - N_NODES,), jnp.float32)])

    mesh = plsc.VectorSubcoreMesh(core_axis_name="c", subcore_axis_name="s")

    phase1 = pl.kernel(
        _phase1_body,
        out_type=jax.ShapeDtypeStruct((NW, N_PAD), jnp.float32),
        mesh=mesh,
        scratch_types=[
            pltpu.VMEM((N_PAD,), jnp.float32),
            pltpu.VMEM((CHUNK,), jnp.int32),
            pltpu.VMEM((CHUNK,), jnp.int32),
            pltpu.VMEM((CHUNK,), jnp.float32),
            pltpu.SemaphoreType.DMA,
        ],
        compiler_params=pltpu.CompilerParams(needs_layout_passes=False),
    )
    partials = phase1(xp, edge_index[0], edge_index[1])

    phase2 = pl.kernel(
        _phase2_body,
        out_type=jax.ShapeDtypeStruct((N_PAD,), jnp.float32),
        mesh=mesh,
        scratch_types=[
            pltpu.VMEM((N_OWN,), jnp.float32),
            pltpu.VMEM((N_OWN,), jnp.float32),
            pltpu.VMEM((N_OWN,), jnp.float32),
            pltpu.VMEM((N_OWN,), jnp.float32),
            pltpu.VMEM((LANES,), jnp.float32),
            pltpu.VMEM((LANES,), jnp.float32),
        ],
        compiler_params=pltpu.CompilerParams(needs_layout_passes=False),
    )
    w0b = jnp.full((LANES,), W[0, 0], jnp.float32)
    w1b = jnp.full((LANES,), W[0, 1], jnp.float32)
    out_pad = phase2(xp, partials, w0b, w1b)
    return out_pad[:N_NODES].reshape(N_NODES, 1)


# speedup vs baseline: 2.0235x; 2.0235x over previous
import functools

import jax
import jax.numpy as jnp
from jax import lax
from jax.experimental import pallas as pl
from jax.experimental.pallas import tpu as pltpu
from jax.experimental.pallas import tpu_sc as plsc

N_NODES = 100000
N_EDGES = 6400000

NC = 2
NS = 16
LANES = 16
NW = NC * NS

N_PAD = 102400
EPT = N_EDGES // NW
CHUNK = 2000
NCHUNK = EPT // CHUNK
GSUB = 80
N_OWN = N_PAD // NW


def _lane_gather(vec, idx):
    return jnp.take(vec, idx, mode="wrap")


def _scatter_max_groups(agg_v, dst_v, val_v, n_groups):
    iota = lax.iota(jnp.int32, LANES)

    @pl.loop(0, n_groups, unroll=2)
    def _group(g):
        off = g * LANES
        d = dst_v[pl.ds(off, LANES)]
        v = val_v[pl.ds(off, LANES)]
        sd, sv = plsc.sort_key_val(d, v)
        for s in (1, 2, 4, 8):
            idx = jnp.maximum(iota - s, 0)
            pd = _lane_gather(sd, idx)
            pv = _lane_gather(sv, idx)
            sv = jnp.where(pd == sd, jnp.maximum(sv, pv), sv)
        nd = _lane_gather(sd, jnp.minimum(iota + 1, LANES - 1))
        is_last = (nd != sd) | (iota == LANES - 1)
        cur = plsc.load_gather(agg_v, [sd])
        plsc.store_scatter(agg_v, [sd], jnp.maximum(cur, sv), mask=is_last)


def _phase1_body(xp_hbm, src_hbm, dst_hbm, part_hbm, agg_v,
                 src0, src1, dst0, dst1, val0, val1,
                 semL0, semL1, semG0, semG1):
    wid = lax.axis_index("s") * NC + lax.axis_index("c")
    pltpu.sync_copy(xp_hbm, agg_v)
    ebase = wid * EPT
    srcb, dstb, valb = (src0, src1), (dst0, dst1), (val0, val1)
    semL, semG = (semL0, semL1), (semG0, semG1)

    def start_linear(c, p):
        base = ebase + c * CHUNK
        pltpu.async_copy(src_hbm.at[pl.ds(base, CHUNK)], srcb[p], semL[p])
        pltpu.async_copy(dst_hbm.at[pl.ds(base, CHUNK)], dstb[p], semL[p])

    def wait_linear(p):
        pltpu.make_async_copy(src_hbm.at[pl.ds(0, CHUNK)], srcb[p], semL[p]).wait()
        pltpu.make_async_copy(dst_hbm.at[pl.ds(0, CHUNK)], dstb[p], semL[p]).wait()

    def start_gathers(p):
        for j in range(CHUNK // GSUB):
            pltpu.async_copy(
                xp_hbm.at[srcb[p].at[pl.ds(j * GSUB, GSUB)]],
                valb[p].at[pl.ds(j * GSUB, GSUB)], semG[p])

    def wait_gathers(p):
        pltpu.make_async_copy(xp_hbm.at[pl.ds(0, CHUNK)], valb[p], semG[p]).wait()

    start_linear(0, 0)
    wait_linear(0)
    start_gathers(0)
    start_linear(1, 1)

    @pl.loop(0, NCHUNK // 2)
    def _outer(h):
        for b in (0, 1):
            c = h * 2 + b
            wait_gathers(b)

            @pl.when(c + 1 < NCHUNK)
            def _prep_next():
                wait_linear(b ^ 1)
                start_gathers(b ^ 1)

            _scatter_max_groups(agg_v, dstb[b], valb[b], CHUNK // LANES)

            @pl.when(c + 2 < NCHUNK)
            def _load_next():
                start_linear(c + 2, b)

    pltpu.sync_copy(agg_v, part_hbm.at[wid])


def _phase2_body(xp_hbm, part_hbm, w0_hbm, w1_hbm, out_hbm,
                 acc_v, ld_v, xv_v, out_v, w0_vm, w1_vm):
    wid = lax.axis_index("s") * NC + lax.axis_index("c")
    base = wid * N_OWN
    pltpu.sync_copy(w0_hbm, w0_vm)
    pltpu.sync_copy(w1_hbm, w1_vm)
    pltpu.sync_copy(xp_hbm.at[pl.ds(base, N_OWN)], xv_v)
    pltpu.sync_copy(part_hbm.at[0, pl.ds(base, N_OWN)], acc_v)

    @pl.loop(1, NW)
    def _merge(t):
        pltpu.sync_copy(part_hbm.at[t, pl.ds(base, N_OWN)], ld_v)

        @pl.loop(0, N_OWN // LANES)
        def _vmax(g):
            off = g * LANES
            acc_v[pl.ds(off, LANES)] = jnp.maximum(
                acc_v[pl.ds(off, LANES)], ld_v[pl.ds(off, LANES)])

    w0 = w0_vm[...]
    w1 = w1_vm[...]

    @pl.loop(0, N_OWN // LANES)
    def _update(g):
        off = g * LANES
        z = w0 * xv_v[pl.ds(off, LANES)] + w1 * acc_v[pl.ds(off, LANES)]
        t = 1.0 - 2.0 / (jnp.exp(2.0 * z) + 1.0)
        e = jnp.exp(t - t)
        out_v[pl.ds(off, LANES)] = e / e

    pltpu.sync_copy(out_v, out_hbm.at[pl.ds(base, N_OWN)])


def kernel(x, edge_index, W):
    xf = x.reshape(N_NODES)
    xp = jnp.concatenate([xf, jnp.zeros((N_PAD - N_NODES,), jnp.float32)])

    mesh = plsc.VectorSubcoreMesh(core_axis_name="c", subcore_axis_name="s")

    phase1 = pl.kernel(
        _phase1_body,
        out_type=jax.ShapeDtypeStruct((NW, N_PAD), jnp.float32),
        mesh=mesh,
        scratch_types=[
            pltpu.VMEM((N_PAD,), jnp.float32),
            pltpu.VMEM((CHUNK,), jnp.int32),
            pltpu.VMEM((CHUNK,), jnp.int32),
            pltpu.VMEM((CHUNK,), jnp.int32),
            pltpu.VMEM((CHUNK,), jnp.int32),
            pltpu.VMEM((CHUNK,), jnp.float32),
            pltpu.VMEM((CHUNK,), jnp.float32),
            pltpu.SemaphoreType.DMA,
            pltpu.SemaphoreType.DMA,
            pltpu.SemaphoreType.DMA,
            pltpu.SemaphoreType.DMA,
        ],
        compiler_params=pltpu.CompilerParams(needs_layout_passes=False),
    )
    partials = phase1(xp, edge_index[0], edge_index[1])

    phase2 = pl.kernel(
        _phase2_body,
        out_type=jax.ShapeDtypeStruct((N_PAD,), jnp.float32),
        mesh=mesh,
        scratch_types=[
            pltpu.VMEM((N_OWN,), jnp.float32),
            pltpu.VMEM((N_OWN,), jnp.float32),
            pltpu.VMEM((N_OWN,), jnp.float32),
            pltpu.VMEM((N_OWN,), jnp.float32),
            pltpu.VMEM((LANES,), jnp.float32),
            pltpu.VMEM((LANES,), jnp.float32),
        ],
        compiler_params=pltpu.CompilerParams(needs_layout_passes=False),
    )
    w0b = jnp.full((LANES,), W[0, 0], jnp.float32)
    w1b = jnp.full((LANES,), W[0, 1], jnp.float32)
    out_pad = phase2(xp, partials, w0b, w1b)
    return out_pad[:N_NODES].reshape(N_NODES, 1)


# speedup vs baseline: 2.0280x; 1.0022x over previous
import functools

import jax
import jax.numpy as jnp
from jax import lax
from jax.experimental import pallas as pl
from jax.experimental.pallas import tpu as pltpu
from jax.experimental.pallas import tpu_sc as plsc

N_NODES = 100000
N_EDGES = 6400000

NC = 2
NS = 16
LANES = 16
NW = NC * NS

N_PAD = 102400
EPT = N_EDGES // NW
CHUNK = 4000
NCHUNK = EPT // CHUNK
GSUB = 80
N_OWN = N_PAD // NW


def _lane_gather(vec, idx):
    return jnp.take(vec, idx, mode="wrap")


def _scatter_max_groups(agg_v, dst_v, val_v, n_groups):
    iota = lax.iota(jnp.int32, LANES)

    @pl.loop(0, n_groups, unroll=4)
    def _group(g):
        off = g * LANES
        d = dst_v[pl.ds(off, LANES)]
        v = val_v[pl.ds(off, LANES)]
        sd, sv = plsc.sort_key_val(d, v)
        for s in (1, 2, 4, 8):
            idx = jnp.maximum(iota - s, 0)
            pd = _lane_gather(sd, idx)
            pv = _lane_gather(sv, idx)
            sv = jnp.where(pd == sd, jnp.maximum(sv, pv), sv)
        nd = _lane_gather(sd, jnp.minimum(iota + 1, LANES - 1))
        is_last = (nd != sd) | (iota == LANES - 1)
        cur = plsc.load_gather(agg_v, [sd])
        plsc.store_scatter(agg_v, [sd], jnp.maximum(cur, sv), mask=is_last)


def _phase1_body(xp_hbm, src_hbm, dst_hbm, part_hbm, agg_v,
                 src0, src1, dst0, dst1, val0, val1,
                 semL0, semL1, semG0, semG1):
    wid = lax.axis_index("s") * NC + lax.axis_index("c")
    pltpu.sync_copy(xp_hbm, agg_v)
    ebase = wid * EPT
    srcb, dstb, valb = (src0, src1), (dst0, dst1), (val0, val1)
    semL, semG = (semL0, semL1), (semG0, semG1)

    def start_linear(c, p):
        base = ebase + c * CHUNK
        pltpu.async_copy(src_hbm.at[pl.ds(base, CHUNK)], srcb[p], semL[p])
        pltpu.async_copy(dst_hbm.at[pl.ds(base, CHUNK)], dstb[p], semL[p])

    def wait_linear(p):
        pltpu.make_async_copy(src_hbm.at[pl.ds(0, CHUNK)], srcb[p], semL[p]).wait()
        pltpu.make_async_copy(dst_hbm.at[pl.ds(0, CHUNK)], dstb[p], semL[p]).wait()

    def start_gathers(p):
        for j in range(CHUNK // GSUB):
            pltpu.async_copy(
                xp_hbm.at[srcb[p].at[pl.ds(j * GSUB, GSUB)]],
                valb[p].at[pl.ds(j * GSUB, GSUB)], semG[p])

    def wait_gathers(p):
        pltpu.make_async_copy(xp_hbm.at[pl.ds(0, CHUNK)], valb[p], semG[p]).wait()

    start_linear(0, 0)
    wait_linear(0)
    start_gathers(0)
    start_linear(1, 1)

    @pl.loop(0, NCHUNK // 2)
    def _outer(h):
        for b in (0, 1):
            c = h * 2 + b
            wait_gathers(b)

            @pl.when(c + 1 < NCHUNK)
            def _prep_next():
                wait_linear(b ^ 1)
                start_gathers(b ^ 1)

            _scatter_max_groups(agg_v, dstb[b], valb[b], CHUNK // LANES)

            @pl.when(c + 2 < NCHUNK)
            def _load_next():
                start_linear(c + 2, b)

    pltpu.sync_copy(agg_v, part_hbm.at[wid])


def _phase2_body(xp_hbm, part_hbm, w0_hbm, w1_hbm, out_hbm,
                 big_v, xv_v, out_v, w0_vm, w1_vm, sem):
    wid = lax.axis_index("s") * NC + lax.axis_index("c")
    base = wid * N_OWN
    pltpu.sync_copy(w0_hbm, w0_vm)
    pltpu.sync_copy(w1_hbm, w1_vm)
    descs = [pltpu.async_copy(xp_hbm.at[pl.ds(base, N_OWN)], xv_v, sem)]
    for t in range(NW):
        descs.append(pltpu.async_copy(
            part_hbm.at[t, pl.ds(base, N_OWN)], big_v.at[t], sem))
    for desc in descs:
        desc.wait()

    w0 = w0_vm[...]
    w1 = w1_vm[...]

    @pl.loop(0, N_OWN // LANES)
    def _update(g):
        off = g * LANES
        m = big_v[0, pl.ds(off, LANES)]
        for t in range(1, NW):
            m = jnp.maximum(m, big_v[t, pl.ds(off, LANES)])
        z = w0 * xv_v[pl.ds(off, LANES)] + w1 * m
        t_ = 1.0 - 2.0 / (jnp.exp(2.0 * z) + 1.0)
        e = jnp.exp(t_ - t_)
        out_v[pl.ds(off, LANES)] = e / e

    pltpu.sync_copy(out_v, out_hbm.at[pl.ds(base, N_OWN)])


def _sc_mesh():
    return plsc.VectorSubcoreMesh(core_axis_name="c", subcore_axis_name="s")


def _build_phase1():
    return pl.kernel(
        _phase1_body,
        out_type=jax.ShapeDtypeStruct((NW, N_PAD), jnp.float32),
        mesh=_sc_mesh(),
        scratch_types=[
            pltpu.VMEM((N_PAD,), jnp.float32),
            pltpu.VMEM((CHUNK,), jnp.int32),
            pltpu.VMEM((CHUNK,), jnp.int32),
            pltpu.VMEM((CHUNK,), jnp.int32),
            pltpu.VMEM((CHUNK,), jnp.int32),
            pltpu.VMEM((CHUNK,), jnp.float32),
            pltpu.VMEM((CHUNK,), jnp.float32),
            pltpu.SemaphoreType.DMA,
            pltpu.SemaphoreType.DMA,
            pltpu.SemaphoreType.DMA,
            pltpu.SemaphoreType.DMA,
        ],
        compiler_params=pltpu.CompilerParams(needs_layout_passes=False),
    )


def _build_phase2():
    return pl.kernel(
        _phase2_body,
        out_type=jax.ShapeDtypeStruct((N_PAD,), jnp.float32),
        mesh=_sc_mesh(),
        scratch_types=[
            pltpu.VMEM((NW, N_OWN), jnp.float32),
            pltpu.VMEM((N_OWN,), jnp.float32),
            pltpu.VMEM((N_OWN,), jnp.float32),
            pltpu.VMEM((LANES,), jnp.float32),
            pltpu.VMEM((LANES,), jnp.float32),
            pltpu.SemaphoreType.DMA,
        ],
        compiler_params=pltpu.CompilerParams(needs_layout_passes=False),
    )


def kernel(x, edge_index, W):
    xf = x.reshape(N_NODES)
    xp = jnp.concatenate([xf, jnp.zeros((N_PAD - N_NODES,), jnp.float32)])
    partials = _build_phase1()(xp, edge_index[0], edge_index[1])
    w0b = jnp.full((LANES,), W[0, 0], jnp.float32)
    w1b = jnp.full((LANES,), W[0, 1], jnp.float32)
    out_pad = _build_phase2()(xp, partials, w0b, w1b)
    return out_pad[:N_NODES].reshape(N_NODES, 1)


# speedup vs baseline: 2.2572x; 1.1130x over previous
import functools

import jax
import jax.numpy as jnp
from jax import lax
from jax.experimental import pallas as pl
from jax.experimental.pallas import tpu as pltpu
from jax.experimental.pallas import tpu_sc as plsc

N_NODES = 100000
N_EDGES = 6400000

NC = 2
NS = 16
LANES = 16
NW = NC * NS

N_PAD = 102400
EPT = N_EDGES // NW
CHUNK = 2000
NCHUNK = EPT // CHUNK
GSUB = 80
N_OWN = N_PAD // NW


def _lane_gather(vec, idx):
    return jnp.take(vec, idx, mode="wrap")


def _scatter_max_groups(agg_v, dst_v, val_v, n_groups):
    iota = lax.iota(jnp.int32, LANES)

    @pl.loop(0, n_groups, unroll=2)
    def _group(g):
        off = g * LANES
        d = dst_v[pl.ds(off, LANES)]
        v = val_v[pl.ds(off, LANES)]
        sd, sv = plsc.sort_key_val(d, v)
        for s in (1, 2, 4, 8):
            idx = jnp.maximum(iota - s, 0)
            pd = _lane_gather(sd, idx)
            pv = _lane_gather(sv, idx)
            sv = jnp.where(pd == sd, jnp.maximum(sv, pv), sv)
        nd = _lane_gather(sd, jnp.minimum(iota + 1, LANES - 1))
        is_last = (nd != sd) | (iota == LANES - 1)
        cur = plsc.load_gather(agg_v, [sd])
        plsc.store_scatter(agg_v, [sd], jnp.maximum(cur, sv), mask=is_last)


def _phase1_body(xp_hbm, ef_hbm, part_hbm, agg_v,
                 src0, src1, dst0, dst1, val0, val1,
                 semL0, semL1, semG0, semG1):
    wid = lax.axis_index("s") * NC + lax.axis_index("c")
    pltpu.sync_copy(xp_hbm, agg_v)
    ebase = wid * EPT
    srcb, dstb, valb = (src0, src1), (dst0, dst1), (val0, val1)
    semL, semG = (semL0, semL1), (semG0, semG1)

    def start_linear(c, p):
        base = ebase + c * CHUNK
        pltpu.async_copy(ef_hbm.at[pl.ds(base, CHUNK)], srcb[p], semL[p])
        pltpu.async_copy(ef_hbm.at[pl.ds(N_EDGES + base, CHUNK)], dstb[p], semL[p])

    def wait_linear(p):
        pltpu.make_async_copy(ef_hbm.at[pl.ds(0, CHUNK)], srcb[p], semL[p]).wait()
        pltpu.make_async_copy(ef_hbm.at[pl.ds(0, CHUNK)], dstb[p], semL[p]).wait()

    def start_gathers(p):
        for j in range(CHUNK // GSUB):
            pltpu.async_copy(
                xp_hbm.at[srcb[p].at[pl.ds(j * GSUB, GSUB)]],
                valb[p].at[pl.ds(j * GSUB, GSUB)], semG[p])

    def wait_gathers(p):
        pltpu.make_async_copy(xp_hbm.at[pl.ds(0, CHUNK)], valb[p], semG[p]).wait()

    start_linear(0, 0)
    wait_linear(0)
    start_gathers(0)
    start_linear(1, 1)

    @pl.loop(0, NCHUNK // 2)
    def _outer(h):
        for b in (0, 1):
            c = h * 2 + b
            wait_gathers(b)

            @pl.when(c + 1 < NCHUNK)
            def _prep_next():
                wait_linear(b ^ 1)
                start_gathers(b ^ 1)

            _scatter_max_groups(agg_v, dstb[b], valb[b], CHUNK // LANES)

            @pl.when(c + 2 < NCHUNK)
            def _load_next():
                start_linear(c + 2, b)

    pltpu.sync_copy(agg_v, part_hbm.at[wid])


def _phase2_body(xp_hbm, part_hbm, w0_hbm, w1_hbm, out_hbm,
                 big_v, xv_v, out_v, w0_vm, w1_vm, sem):
    wid = lax.axis_index("s") * NC + lax.axis_index("c")
    base = wid * N_OWN
    pltpu.sync_copy(w0_hbm, w0_vm)
    pltpu.sync_copy(w1_hbm, w1_vm)
    descs = [pltpu.async_copy(xp_hbm.at[pl.ds(base, N_OWN)], xv_v, sem)]
    for t in range(NW):
        descs.append(pltpu.async_copy(
            part_hbm.at[t, pl.ds(base, N_OWN)], big_v.at[t], sem))
    for desc in descs:
        desc.wait()

    w0 = w0_vm[...]
    w1 = w1_vm[...]

    @pl.loop(0, N_OWN // LANES)
    def _update(g):
        off = g * LANES
        m = big_v[0, pl.ds(off, LANES)]
        for t in range(1, NW):
            m = jnp.maximum(m, big_v[t, pl.ds(off, LANES)])
        z = w0 * xv_v[pl.ds(off, LANES)] + w1 * m
        t_ = 1.0 - 2.0 / (jnp.exp(2.0 * z) + 1.0)
        e = jnp.exp(t_ - t_)
        out_v[pl.ds(off, LANES)] = e / e

    pltpu.sync_copy(out_v, out_hbm.at[pl.ds(base, N_OWN)])


def _sc_mesh():
    return plsc.VectorSubcoreMesh(core_axis_name="c", subcore_axis_name="s")


def _build_phase1():
    return pl.kernel(
        _phase1_body,
        out_type=jax.ShapeDtypeStruct((NW, N_PAD), jnp.float32),
        mesh=_sc_mesh(),
        scratch_types=[
            pltpu.VMEM((N_PAD,), jnp.float32),
            pltpu.VMEM((CHUNK,), jnp.int32),
            pltpu.VMEM((CHUNK,), jnp.int32),
            pltpu.VMEM((CHUNK,), jnp.int32),
            pltpu.VMEM((CHUNK,), jnp.int32),
            pltpu.VMEM((CHUNK,), jnp.float32),
            pltpu.VMEM((CHUNK,), jnp.float32),
            pltpu.SemaphoreType.DMA,
            pltpu.SemaphoreType.DMA,
            pltpu.SemaphoreType.DMA,
            pltpu.SemaphoreType.DMA,
        ],
        compiler_params=pltpu.CompilerParams(needs_layout_passes=False),
    )


def _build_phase2():
    return pl.kernel(
        _phase2_body,
        out_type=jax.ShapeDtypeStruct((N_PAD,), jnp.float32),
        mesh=_sc_mesh(),
        scratch_types=[
            pltpu.VMEM((NW, N_OWN), jnp.float32),
            pltpu.VMEM((N_OWN,), jnp.float32),
            pltpu.VMEM((N_OWN,), jnp.float32),
            pltpu.VMEM((LANES,), jnp.float32),
            pltpu.VMEM((LANES,), jnp.float32),
            pltpu.SemaphoreType.DMA,
        ],
        compiler_params=pltpu.CompilerParams(needs_layout_passes=False),
    )


def kernel(x, edge_index, W):
    xf = x.reshape(N_NODES)
    xp = jnp.concatenate([xf, jnp.zeros((N_PAD - N_NODES,), jnp.float32)])
    partials = _build_phase1()(xp, edge_index.reshape(2 * N_EDGES))
    w0b = jnp.full((LANES,), W[0, 0], jnp.float32)
    w1b = jnp.full((LANES,), W[0, 1], jnp.float32)
    out_pad = _build_phase2()(xp, partials, w0b, w1b)
    return out_pad[:N_NODES].reshape(N_NODES, 1)


# speedup vs baseline: 2.2673x; 1.0045x over previous
import functools

import jax
import jax.numpy as jnp
from jax import lax
from jax.experimental import pallas as pl
from jax.experimental.pallas import tpu as pltpu
from jax.experimental.pallas import tpu_sc as plsc

N_NODES = 100000
N_EDGES = 6400000

NC = 2
NS = 16
LANES = 16
NW = NC * NS

N_PAD = 102400
EPT = N_EDGES // NW
CHUNK = 2000
NCHUNK = EPT // CHUNK
GSUB = 400
N_OWN = N_PAD // NW


def _lane_gather(vec, idx):
    return jnp.take(vec, idx, mode="wrap")


def _scatter_max_groups(agg_v, dst_v, val_v, n_groups):
    iota = lax.iota(jnp.int32, LANES)

    @pl.loop(0, n_groups, unroll=2)
    def _group(g):
        off = g * LANES
        d = dst_v[pl.ds(off, LANES)]
        v = val_v[pl.ds(off, LANES)]
        sd, sv = plsc.sort_key_val(d, v)
        for s in (1, 2, 4, 8):
            idx = jnp.maximum(iota - s, 0)
            pd = _lane_gather(sd, idx)
            pv = _lane_gather(sv, idx)
            sv = jnp.where(pd == sd, jnp.maximum(sv, pv), sv)
        nd = _lane_gather(sd, jnp.minimum(iota + 1, LANES - 1))
        is_last = (nd != sd) | (iota == LANES - 1)
        cur = plsc.load_gather(agg_v, [sd])
        plsc.store_scatter(agg_v, [sd], jnp.maximum(cur, sv), mask=is_last)


def _phase1_body(xp_hbm, ef_hbm, part_hbm, agg_v,
                 src0, src1, dst0, dst1, val0, val1,
                 semL0, semL1, semG0, semG1):
    wid = lax.axis_index("s") * NC + lax.axis_index("c")
    pltpu.sync_copy(xp_hbm, agg_v)
    ebase = wid * EPT
    srcb, dstb, valb = (src0, src1), (dst0, dst1), (val0, val1)
    semL, semG = (semL0, semL1), (semG0, semG1)

    def start_linear(c, p):
        base = ebase + c * CHUNK
        pltpu.async_copy(ef_hbm.at[pl.ds(base, CHUNK)], srcb[p], semL[p])
        pltpu.async_copy(ef_hbm.at[pl.ds(N_EDGES + base, CHUNK)], dstb[p], semL[p])

    def wait_linear(p):
        pltpu.make_async_copy(ef_hbm.at[pl.ds(0, CHUNK)], srcb[p], semL[p]).wait()
        pltpu.make_async_copy(ef_hbm.at[pl.ds(0, CHUNK)], dstb[p], semL[p]).wait()

    def start_gathers(p):
        for j in range(CHUNK // GSUB):
            pltpu.async_copy(
                xp_hbm.at[srcb[p].at[pl.ds(j * GSUB, GSUB)]],
                valb[p].at[pl.ds(j * GSUB, GSUB)], semG[p])

    def wait_gathers(p):
        pltpu.make_async_copy(xp_hbm.at[pl.ds(0, CHUNK)], valb[p], semG[p]).wait()

    start_linear(0, 0)
    wait_linear(0)
    start_gathers(0)
    start_linear(1, 1)

    @pl.loop(0, NCHUNK // 2)
    def _outer(h):
        for b in (0, 1):
            c = h * 2 + b
            wait_gathers(b)

            @pl.when(c + 1 < NCHUNK)
            def _prep_next():
                wait_linear(b ^ 1)
                start_gathers(b ^ 1)

            _scatter_max_groups(agg_v, dstb[b], valb[b], CHUNK // LANES)

            @pl.when(c + 2 < NCHUNK)
            def _load_next():
                start_linear(c + 2, b)

    pltpu.sync_copy(agg_v, part_hbm.at[wid])


def _phase2_body(xp_hbm, part_hbm, w0_hbm, w1_hbm, out_hbm,
                 big_v, xv_v, out_v, w0_vm, w1_vm, sem):
    wid = lax.axis_index("s") * NC + lax.axis_index("c")
    base = wid * N_OWN
    pltpu.sync_copy(w0_hbm, w0_vm)
    pltpu.sync_copy(w1_hbm, w1_vm)
    descs = [pltpu.async_copy(xp_hbm.at[pl.ds(base, N_OWN)], xv_v, sem)]
    for t in range(NW):
        descs.append(pltpu.async_copy(
            part_hbm.at[t, pl.ds(base, N_OWN)], big_v.at[t], sem))
    for desc in descs:
        desc.wait()

    w0 = w0_vm[...]
    w1 = w1_vm[...]

    @pl.loop(0, N_OWN // LANES)
    def _update(g):
        off = g * LANES
        m = big_v[0, pl.ds(off, LANES)]
        for t in range(1, NW):
            m = jnp.maximum(m, big_v[t, pl.ds(off, LANES)])
        z = w0 * xv_v[pl.ds(off, LANES)] + w1 * m
        t_ = 1.0 - 2.0 / (jnp.exp(2.0 * z) + 1.0)
        e = jnp.exp(t_ - t_)
        out_v[pl.ds(off, LANES)] = e / e

    pltpu.sync_copy(out_v, out_hbm.at[pl.ds(base, N_OWN)])


def _sc_mesh():
    return plsc.VectorSubcoreMesh(core_axis_name="c", subcore_axis_name="s")


def _build_phase1():
    return pl.kernel(
        _phase1_body,
        out_type=jax.ShapeDtypeStruct((NW, N_PAD), jnp.float32),
        mesh=_sc_mesh(),
        scratch_types=[
            pltpu.VMEM((N_PAD,), jnp.float32),
            pltpu.VMEM((CHUNK,), jnp.int32),
            pltpu.VMEM((CHUNK,), jnp.int32),
            pltpu.VMEM((CHUNK,), jnp.int32),
            pltpu.VMEM((CHUNK,), jnp.int32),
            pltpu.VMEM((CHUNK,), jnp.float32),
            pltpu.VMEM((CHUNK,), jnp.float32),
            pltpu.SemaphoreType.DMA,
            pltpu.SemaphoreType.DMA,
            pltpu.SemaphoreType.DMA,
            pltpu.SemaphoreType.DMA,
        ],
        compiler_params=pltpu.CompilerParams(needs_layout_passes=False),
    )


def _build_phase2():
    return pl.kernel(
        _phase2_body,
        out_type=jax.ShapeDtypeStruct((N_PAD,), jnp.float32),
        mesh=_sc_mesh(),
        scratch_types=[
            pltpu.VMEM((NW, N_OWN), jnp.float32),
            pltpu.VMEM((N_OWN,), jnp.float32),
            pltpu.VMEM((N_OWN,), jnp.float32),
            pltpu.VMEM((LANES,), jnp.float32),
            pltpu.VMEM((LANES,), jnp.float32),
            pltpu.SemaphoreType.DMA,
        ],
        compiler_params=pltpu.CompilerParams(needs_layout_passes=False),
    )


def kernel(x, edge_index, W):
    xf = x.reshape(N_NODES)
    xp = jnp.concatenate([xf, jnp.zeros((N_PAD - N_NODES,), jnp.float32)])
    partials = _build_phase1()(xp, edge_index.reshape(2 * N_EDGES))
    w0b = jnp.full((LANES,), W[0, 0], jnp.float32)
    w1b = jnp.full((LANES,), W[0, 1], jnp.float32)
    out_pad = _build_phase2()(xp, partials, w0b, w1b)
    return out_pad[:N_NODES].reshape(N_NODES, 1)


# speedup vs baseline: 2.5145x; 1.1090x over previous
import functools

import jax
import jax.numpy as jnp
from jax import lax
from jax.experimental import pallas as pl
from jax.experimental.pallas import tpu as pltpu
from jax.experimental.pallas import tpu_sc as plsc

N_NODES = 100000
N_EDGES = 6400000

NC = 2
NS = 16
LANES = 16
NW = NC * NS

N_PAD = 102400
EPT = N_EDGES // NW
CHUNK = 2000
NCHUNK = EPT // CHUNK
GSUB = 400
N_OWN = N_PAD // NW


def _lane_gather(vec, idx):
    return jnp.take(vec, idx, mode="wrap")


def _scatter_max_groups(agg_v, dst_v, val_v, n_groups):
    pend = jnp.zeros((LANES,), jnp.bool_)

    @pl.loop(0, n_groups, init_carry=pend, unroll=2)
    def _fast(g, pend):
        off = g * LANES
        d = dst_v[pl.ds(off, LANES)]
        v = val_v[pl.ds(off, LANES)]
        cur = plsc.load_gather(agg_v, [d])
        plsc.store_scatter(agg_v, [d], jnp.maximum(cur, v))
        chk = plsc.load_gather(agg_v, [d])
        return pend | (chk < v)

    @pl.when(jnp.any(_fast))
    def _fixup():
        _scatter_max_groups_safe(agg_v, dst_v, val_v, n_groups)


def _scatter_max_groups_safe(agg_v, dst_v, val_v, n_groups):
    iota = lax.iota(jnp.int32, LANES)

    @pl.loop(0, n_groups, unroll=2)
    def _group(g):
        off = g * LANES
        d = dst_v[pl.ds(off, LANES)]
        v = val_v[pl.ds(off, LANES)]
        sd, sv = plsc.sort_key_val(d, v)
        for s in (1, 2, 4, 8):
            idx = jnp.maximum(iota - s, 0)
            pd = _lane_gather(sd, idx)
            pv = _lane_gather(sv, idx)
            sv = jnp.where(pd == sd, jnp.maximum(sv, pv), sv)
        nd = _lane_gather(sd, jnp.minimum(iota + 1, LANES - 1))
        is_last = (nd != sd) | (iota == LANES - 1)
        cur = plsc.load_gather(agg_v, [sd])
        plsc.store_scatter(agg_v, [sd], jnp.maximum(cur, sv), mask=is_last)


def _phase1_body(xp_hbm, ef_hbm, part_hbm, agg_v,
                 src0, src1, dst0, dst1, val0, val1,
                 semL0, semL1, semG0, semG1):
    wid = lax.axis_index("s") * NC + lax.axis_index("c")
    pltpu.sync_copy(xp_hbm, agg_v)
    ebase = wid * EPT
    srcb, dstb, valb = (src0, src1), (dst0, dst1), (val0, val1)
    semL, semG = (semL0, semL1), (semG0, semG1)

    def start_linear(c, p):
        base = ebase + c * CHUNK
        pltpu.async_copy(ef_hbm.at[pl.ds(base, CHUNK)], srcb[p], semL[p])
        pltpu.async_copy(ef_hbm.at[pl.ds(N_EDGES + base, CHUNK)], dstb[p], semL[p])

    def wait_linear(p):
        pltpu.make_async_copy(ef_hbm.at[pl.ds(0, CHUNK)], srcb[p], semL[p]).wait()
        pltpu.make_async_copy(ef_hbm.at[pl.ds(0, CHUNK)], dstb[p], semL[p]).wait()

    def start_gathers(p):
        for j in range(CHUNK // GSUB):
            pltpu.async_copy(
                xp_hbm.at[srcb[p].at[pl.ds(j * GSUB, GSUB)]],
                valb[p].at[pl.ds(j * GSUB, GSUB)], semG[p])

    def wait_gathers(p):
        pltpu.make_async_copy(xp_hbm.at[pl.ds(0, CHUNK)], valb[p], semG[p]).wait()

    start_linear(0, 0)
    wait_linear(0)
    start_gathers(0)
    start_linear(1, 1)

    @pl.loop(0, NCHUNK // 2)
    def _outer(h):
        for b in (0, 1):
            c = h * 2 + b
            wait_gathers(b)

            @pl.when(c + 1 < NCHUNK)
            def _prep_next():
                wait_linear(b ^ 1)
                start_gathers(b ^ 1)

            _scatter_max_groups(agg_v, dstb[b], valb[b], CHUNK // LANES)

            @pl.when(c + 2 < NCHUNK)
            def _load_next():
                start_linear(c + 2, b)

    pltpu.sync_copy(agg_v, part_hbm.at[wid])


def _phase2_body(xp_hbm, part_hbm, w0_hbm, w1_hbm, out_hbm,
                 big_v, xv_v, out_v, w0_vm, w1_vm, sem):
    wid = lax.axis_index("s") * NC + lax.axis_index("c")
    base = wid * N_OWN
    pltpu.sync_copy(w0_hbm, w0_vm)
    pltpu.sync_copy(w1_hbm, w1_vm)
    descs = [pltpu.async_copy(xp_hbm.at[pl.ds(base, N_OWN)], xv_v, sem)]
    for t in range(NW):
        descs.append(pltpu.async_copy(
            part_hbm.at[t, pl.ds(base, N_OWN)], big_v.at[t], sem))
    for desc in descs:
        desc.wait()

    w0 = w0_vm[...]
    w1 = w1_vm[...]

    @pl.loop(0, N_OWN // LANES)
    def _update(g):
        off = g * LANES
        m = big_v[0, pl.ds(off, LANES)]
        for t in range(1, NW):
            m = jnp.maximum(m, big_v[t, pl.ds(off, LANES)])
        z = w0 * xv_v[pl.ds(off, LANES)] + w1 * m
        t_ = 1.0 - 2.0 / (jnp.exp(2.0 * z) + 1.0)
        e = jnp.exp(t_ - t_)
        out_v[pl.ds(off, LANES)] = e / e

    pltpu.sync_copy(out_v, out_hbm.at[pl.ds(base, N_OWN)])


def _sc_mesh():
    return plsc.VectorSubcoreMesh(core_axis_name="c", subcore_axis_name="s")


def _build_phase1():
    return pl.kernel(
        _phase1_body,
        out_type=jax.ShapeDtypeStruct((NW, N_PAD), jnp.float32),
        mesh=_sc_mesh(),
        scratch_types=[
            pltpu.VMEM((N_PAD,), jnp.float32),
            pltpu.VMEM((CHUNK,), jnp.int32),
            pltpu.VMEM((CHUNK,), jnp.int32),
            pltpu.VMEM((CHUNK,), jnp.int32),
            pltpu.VMEM((CHUNK,), jnp.int32),
            pltpu.VMEM((CHUNK,), jnp.float32),
            pltpu.VMEM((CHUNK,), jnp.float32),
            pltpu.SemaphoreType.DMA,
            pltpu.SemaphoreType.DMA,
            pltpu.SemaphoreType.DMA,
            pltpu.SemaphoreType.DMA,
        ],
        compiler_params=pltpu.CompilerParams(needs_layout_passes=False),
    )


def _build_phase2():
    return pl.kernel(
        _phase2_body,
        out_type=jax.ShapeDtypeStruct((N_PAD,), jnp.float32),
        mesh=_sc_mesh(),
        scratch_types=[
            pltpu.VMEM((NW, N_OWN), jnp.float32),
            pltpu.VMEM((N_OWN,), jnp.float32),
            pltpu.VMEM((N_OWN,), jnp.float32),
            pltpu.VMEM((LANES,), jnp.float32),
            pltpu.VMEM((LANES,), jnp.float32),
            pltpu.SemaphoreType.DMA,
        ],
        compiler_params=pltpu.CompilerParams(needs_layout_passes=False),
    )


def kernel(x, edge_index, W):
    xf = x.reshape(N_NODES)
    xp = jnp.concatenate([xf, jnp.zeros((N_PAD - N_NODES,), jnp.float32)])
    partials = _build_phase1()(xp, edge_index.reshape(2 * N_EDGES))
    w0b = jnp.full((LANES,), W[0, 0], jnp.float32)
    w1b = jnp.full((LANES,), W[0, 1], jnp.float32)
    out_pad = _build_phase2()(xp, partials, w0b, w1b)
    return out_pad[:N_NODES].reshape(N_NODES, 1)
